# baseline (speedup 1.0000x reference)
import jax
import jax.numpy as jnp
from jax.experimental import pallas as pl
from jax.experimental.pallas import tpu as pltpu

_KNN = 5
_BOUND_LAMBDA = 1.0
_MAX_STEPS = 100
_NEG_BIG = -3.0e38


def _lame_kernel(scores_ref, feats_ref, out_ref, w_ref, unary_ref, y_ref):
    f = feats_ref[:]
    n = jnp.sqrt(jnp.sum(f * f, axis=1, keepdims=True))
    f = f / jnp.clip(n, 1e-12, None)

    G = jax.lax.dot_general(
        f, f, (((1,), (1,)), ((), ())), preferred_element_type=jnp.float32
    )
    N = G.shape[0]
    row_ids = jax.lax.broadcasted_iota(jnp.int32, (N, N), 0)
    col_ids = jax.lax.broadcasted_iota(jnp.int32, (N, N), 1)
    g = jnp.where(row_ids == col_ids, _NEG_BIG, G)

    for _ in range(_KNN):
        idx = jnp.argmax(g, axis=1, keepdims=True)
        hit = col_ids == idx
        g = jnp.where(hit, _NEG_BIG, g)
    w_ref[:] = jnp.where(
        jnp.logical_and(g == _NEG_BIG, row_ids != col_ids), 1.0, 0.0
    )

    unary = -jnp.log(scores_ref[:] + 1e-10)
    unary_ref[:] = unary
    m0 = jnp.max(-unary, axis=1, keepdims=True)
    e0 = jnp.exp(-unary - m0)
    y_ref[:] = e0 / jnp.sum(e0, axis=1, keepdims=True)

    def cond_fn(state):
        i, _, done = state
        return jnp.logical_and(i < _MAX_STEPS, jnp.logical_not(done))

    def body_fn(state):
        i, oldE, _ = state
        Y = y_ref[:]
        lses = []
        half = N // 2
        for h in range(2):
            r = pl.ds(h * half, half)
            z = _BOUND_LAMBDA * jnp.dot(
                w_ref[r, :], Y, preferred_element_type=jnp.float32
            ) - unary_ref[r, :]
            m = jnp.max(z, axis=1, keepdims=True)
            e = jnp.exp(z - m)
            s = jnp.sum(e, axis=1, keepdims=True)
            y_ref[r, :] = e / s
            lses.append(jnp.sum(m + jnp.log(s)))
        E = -(lses[0] + lses[1])
        done = jnp.logical_and(i > 1, jnp.abs(E - oldE) <= 1e-08 * jnp.abs(oldE))
        return (i + 1, E, done)

    state0 = (jnp.int32(0), jnp.array(jnp.inf, dtype=jnp.float32), jnp.array(False))
    jax.lax.while_loop(cond_fn, body_fn, state0)
    out_ref[:] = y_ref[:]


def kernel(scores_raw, feats):
    B, C, H, Wd = scores_raw.shape
    scores = scores_raw.reshape(-1, H * Wd)
    f = feats.reshape(feats.shape[:-3] + (-1,))
    if f.shape[0] == 1:
        f = jnp.squeeze(f, 0)
    M, L = scores.shape
    return pl.pallas_call(
        _lame_kernel,
        out_shape=jax.ShapeDtypeStruct((M, L), jnp.float32),
        scratch_shapes=[
            pltpu.VMEM((M, M), jnp.float32),
            pltpu.VMEM((M, L), jnp.float32),
            pltpu.VMEM((M, L), jnp.float32),
        ],
    )(scores, f)


# speedup vs baseline: 1.0106x; 1.0106x over previous
import jax
import jax.numpy as jnp
from jax.experimental import pallas as pl
from jax.experimental.pallas import tpu as pltpu

_KNN = 5
_BOUND_LAMBDA = 1.0
_MAX_STEPS = 100
_NEG_BIG = -3.0e38
_DIAG_BIG = -2.0e38


def _lame_kernel(scores_ref, feats_ref, out_ref, w_ref, unary_ref, y_ref):
    f = feats_ref[:]
    n = jnp.sqrt(jnp.sum(f * f, axis=1, keepdims=True))
    f = f / jnp.clip(n, 1e-12, None)

    G = jax.lax.dot_general(
        f, f, (((1,), (1,)), ((), ())), preferred_element_type=jnp.float32
    )
    N = G.shape[0]
    row_ids = jax.lax.broadcasted_iota(jnp.int32, (N, N), 0)
    col_ids = jax.lax.broadcasted_iota(jnp.int32, (N, N), 1)
    g = jnp.where(row_ids == col_ids, _DIAG_BIG, G)

    for _ in range(_KNN):
        idx = jnp.argmax(g, axis=1, keepdims=True)
        hit = col_ids == idx
        g = jnp.where(hit, _NEG_BIG, g)
    w_ref[:] = jnp.where(g == _NEG_BIG, 1.0, 0.0)

    unary = -jnp.log(scores_ref[:] + 1e-10)
    unary_ref[:] = unary
    m0 = jnp.max(-unary, axis=1, keepdims=True)
    e0 = jnp.exp(-unary - m0)
    y_ref[:] = e0 / jnp.sum(e0, axis=1, keepdims=True)

    def cond_fn(state):
        i, _, done = state
        return jnp.logical_and(i < _MAX_STEPS, jnp.logical_not(done))

    def body_fn(state):
        i, oldE, _ = state
        z = _BOUND_LAMBDA * jnp.dot(
            w_ref[:], y_ref[:], preferred_element_type=jnp.float32
        ) - unary_ref[:]
        m = jnp.max(z, axis=1, keepdims=True)
        e = jnp.exp(z - m)
        s = jnp.sum(e, axis=1, keepdims=True)
        y_ref[:] = e / s
        E = -jnp.sum(m + jnp.log(s))
        done = jnp.logical_and(i > 1, jnp.abs(E - oldE) <= 1e-08 * jnp.abs(oldE))
        return (i + 1, E, done)

    state0 = (jnp.int32(0), jnp.array(jnp.inf, dtype=jnp.float32), jnp.array(False))
    jax.lax.while_loop(cond_fn, body_fn, state0)
    out_ref[:] = y_ref[:]


def kernel(scores_raw, feats):
    B, C, H, Wd = scores_raw.shape
    scores = scores_raw.reshape(-1, H * Wd)
    f = feats.reshape(feats.shape[:-3] + (-1,))
    if f.shape[0] == 1:
        f = jnp.squeeze(f, 0)
    M, L = scores.shape
    return pl.pallas_call(
        _lame_kernel,
        out_shape=jax.ShapeDtypeStruct((M, L), jnp.float32),
        scratch_shapes=[
            pltpu.VMEM((M, M), jnp.float32),
            pltpu.VMEM((M, L), jnp.float32),
            pltpu.VMEM((M, L), jnp.float32),
        ],
    )(scores, f)
